# Initial kernel scaffold; baseline (speedup 1.0000x reference)
import jax
import jax.numpy as jnp
from jax.experimental import pallas as pl


def kernel(logits, prop_adj, struct_feat):
    raise NotImplementedError("write your pallas kernel here")


# speedup vs baseline: 1.2990x; 1.2990x over previous
import functools

import jax
import jax.numpy as jnp
import numpy as np
from jax.experimental import pallas as pl

_PROP_STEPS = 2
_ALPHA = 0.2
_GLOBAL_BETA = 0.05
_MIN_ANCHOR = 0.6
_RESIDUAL_SCALE = 0.15
_DEGREE_BIAS = 0.25
_CLUSTERING_BIAS = 0.2
_GRAPH_SCALE_BIAS = 1.0
_SOURCE_CONF_CENTER = 0.55
_SOURCE_CONF_SHARPNESS = 8.0
_RECIPIENT_CONF_CENTER = 0.5
_RECIPIENT_CONF_SHARPNESS = 8.0
_ACCEPT_SHARPNESS = 12.0
_ACCEPT_QUALITY_WEIGHT = 0.7
_ACCEPT_MARGIN_WEIGHT = 0.2
_ACCEPT_STRUCT_WEIGHT = 0.1
_EPS = 1e-8

_C = 64
_BM = 400
_RHS = 128


def _confidence(state):
    score_mass = state.sum(axis=1, keepdims=True)
    norm_scores = state / (score_mass + _EPS)
    max_entropy = float(np.log(state.shape[1]))
    entropy = -(norm_scores * jnp.log(norm_scores + _EPS)).sum(axis=1, keepdims=True)
    certainty = 1.0 - entropy / max_entropy
    mass_scale = jnp.clip(score_mass.mean(), _EPS, None)
    magnitude = jnp.tanh(score_mass / mass_scale)
    return jnp.clip(0.5 * certainty + 0.5 * magnitude, 0.0, 1.0)


def _step_body(adj_ref, b_ref, prop_ref, seed_ref, tb_ref, s_ref, out_ref):
    acc = jnp.dot(adj_ref[...], b_ref[...], preferred_element_type=jnp.float32)
    num = acc[:, :_C]
    den = jnp.maximum(acc[:, _C:_C + 1], _EPS)
    lc = num / den

    prop = prop_ref[...]
    na = s_ref[:, 0:1]
    margin_term = s_ref[:, 1:2]
    rate_base = s_ref[:, 2:3]
    tcoef = s_ref[:, 3:4]
    res_coef = s_ref[:, 4:5]

    dot = jnp.sum(prop * lc, axis=1, keepdims=True)
    nb = jnp.maximum(jnp.sqrt(jnp.sum(lc * lc, axis=1, keepdims=True)), 1e-8)
    local_quality = jnp.clip((dot / (na * nb) + 1.0) * 0.5, 0.0, 1.0)
    quality = _ACCEPT_QUALITY_WEIGHT * local_quality + margin_term
    accept = jax.nn.sigmoid(_ACCEPT_SHARPNESS * quality)
    step_rate = jnp.minimum(rate_base * accept, 1.0)

    target = tb_ref[...] + tcoef * lc
    p = prop + step_rate * (target - prop)
    p = p + res_coef * (seed_ref[...] - p)
    out_ref[...] = p


@functools.partial(jax.jit, static_argnames=())
def _propagate_step(adj, b, prop, seed, target_base, svec):
    n = adj.shape[0]
    return pl.pallas_call(
        _step_body,
        grid=(n // _BM,),
        in_specs=[
            pl.BlockSpec((_BM, n), lambda i: (i, 0)),
            pl.BlockSpec((n, _RHS), lambda i: (0, 0)),
            pl.BlockSpec((_BM, _C), lambda i: (i, 0)),
            pl.BlockSpec((_BM, _C), lambda i: (i, 0)),
            pl.BlockSpec((_BM, _C), lambda i: (i, 0)),
            pl.BlockSpec((_BM, 8), lambda i: (i, 0)),
        ],
        out_specs=pl.BlockSpec((_BM, _C), lambda i: (i, 0)),
        out_shape=jax.ShapeDtypeStruct((n, _C), jnp.float32),
    )(adj, b, prop, seed, target_base, svec)


def kernel(logits, prop_adj, struct_feat):
    n = logits.shape[0]
    seed = jax.nn.relu(logits)
    conf0 = _confidence(seed)
    weighted_seed = conf0 * seed
    global_prior = weighted_seed.sum(axis=0, keepdims=True) / jnp.clip(
        conf0.sum(), _EPS, None)
    anchor = jnp.clip(_MIN_ANCHOR + _ALPHA * conf0, 0.0, 0.995)
    uncertainty = 1.0 - conf0
    log_degree = struct_feat[:, :1]
    low_degree = jnp.clip(1.0 - log_degree, 0.0, 1.0)
    clustering = struct_feat[:, 1:2]
    low_clustering = jnp.clip(1.0 - clustering, 0.0, 1.0)
    graph_scale = jnp.clip(1.0 - clustering.mean(), 0.2, 1.0)
    struct_boost = 1.0 + _DEGREE_BIAS * low_degree + _CLUSTERING_BIAS * low_clustering

    tcoef = (1.0 - anchor) * (1.0 - _GLOBAL_BETA)
    target_base = anchor * seed + (1.0 - anchor) * _GLOBAL_BETA * global_prior
    res_coef = _RESIDUAL_SCALE * uncertainty
    rate_fixed = _GRAPH_SCALE_BIAS * graph_scale * struct_boost * uncertainty

    prop = seed
    conf = conf0
    for _ in range(_PROP_STEPS):
        source_gate = jax.nn.sigmoid(
            _SOURCE_CONF_SHARPNESS * (conf - _SOURCE_CONF_CENTER))
        recipient_gate = jax.nn.sigmoid(
            _RECIPIENT_CONF_SHARPNESS * (_RECIPIENT_CONF_CENTER - conf))
        b = jnp.concatenate(
            [source_gate * prop, source_gate,
             jnp.zeros((n, _RHS - _C - 1), dtype=jnp.float32)], axis=1)

        na = jnp.maximum(
            jnp.sqrt(jnp.sum(prop * prop, axis=1, keepdims=True)), 1e-8)
        probs = prop / (prop.sum(axis=1, keepdims=True) + _EPS)
        topk, _ = jax.lax.top_k(probs, 2)
        margin = topk[:, :1] - topk[:, 1:2]
        margin_term = (_ACCEPT_MARGIN_WEIGHT * margin
                       + _ACCEPT_STRUCT_WEIGHT * clustering)
        rate_base = rate_fixed * recipient_gate

        svec = jnp.concatenate(
            [na, margin_term, rate_base, tcoef, res_coef,
             jnp.zeros((n, 3), dtype=jnp.float32)], axis=1)

        prop = _propagate_step(prop_adj, b, prop, seed, target_base, svec)
        conf = _confidence(prop)

    return prop


# speedup vs baseline: 1.3039x; 1.0038x over previous
import functools

import jax
import jax.numpy as jnp
import numpy as np
from jax.experimental import pallas as pl

_PROP_STEPS = 2
_ALPHA = 0.2
_GLOBAL_BETA = 0.05
_MIN_ANCHOR = 0.6
_RESIDUAL_SCALE = 0.15
_DEGREE_BIAS = 0.25
_CLUSTERING_BIAS = 0.2
_GRAPH_SCALE_BIAS = 1.0
_SOURCE_CONF_CENTER = 0.55
_SOURCE_CONF_SHARPNESS = 8.0
_RECIPIENT_CONF_CENTER = 0.5
_RECIPIENT_CONF_SHARPNESS = 8.0
_ACCEPT_SHARPNESS = 12.0
_ACCEPT_QUALITY_WEIGHT = 0.7
_ACCEPT_MARGIN_WEIGHT = 0.2
_ACCEPT_STRUCT_WEIGHT = 0.1
_EPS = 1e-8

_C = 64
_BM = 400
_RHS = 128


def _confidence(state):
    score_mass = state.sum(axis=1, keepdims=True)
    norm_scores = state / (score_mass + _EPS)
    max_entropy = float(np.log(state.shape[1]))
    entropy = -(norm_scores * jnp.log(norm_scores + _EPS)).sum(axis=1, keepdims=True)
    certainty = 1.0 - entropy / max_entropy
    mass_scale = jnp.clip(score_mass.mean(), _EPS, None)
    magnitude = jnp.tanh(score_mass / mass_scale)
    return jnp.clip(0.5 * certainty + 0.5 * magnitude, 0.0, 1.0)


def _step_body(adj_ref, b_ref, prop_ref, seed_ref, tb_ref, s_ref, out_ref):
    acc = jnp.dot(adj_ref[...].astype(jnp.bfloat16), b_ref[...],
                  preferred_element_type=jnp.float32)
    num = acc[:, :_C]
    den = jnp.maximum(acc[:, _C:_C + 1], _EPS)
    lc = num / den

    prop = prop_ref[...]
    na = s_ref[:, 0:1]
    margin_term = s_ref[:, 1:2]
    rate_base = s_ref[:, 2:3]
    tcoef = s_ref[:, 3:4]
    res_coef = s_ref[:, 4:5]

    dot = jnp.sum(prop * lc, axis=1, keepdims=True)
    nb = jnp.maximum(jnp.sqrt(jnp.sum(lc * lc, axis=1, keepdims=True)), 1e-8)
    local_quality = jnp.clip((dot / (na * nb) + 1.0) * 0.5, 0.0, 1.0)
    quality = _ACCEPT_QUALITY_WEIGHT * local_quality + margin_term
    accept = jax.nn.sigmoid(_ACCEPT_SHARPNESS * quality)
    step_rate = jnp.minimum(rate_base * accept, 1.0)

    target = tb_ref[...] + tcoef * lc
    p = prop + step_rate * (target - prop)
    p = p + res_coef * (seed_ref[...] - p)
    out_ref[...] = p


@functools.partial(jax.jit, static_argnames=())
def _propagate_step(adj, b, prop, seed, target_base, svec):
    n = adj.shape[0]
    return pl.pallas_call(
        _step_body,
        grid=(n // _BM,),
        in_specs=[
            pl.BlockSpec((_BM, n), lambda i: (i, 0)),
            pl.BlockSpec((n, _RHS), lambda i: (0, 0)),

            pl.BlockSpec((_BM, _C), lambda i: (i, 0)),
            pl.BlockSpec((_BM, _C), lambda i: (i, 0)),
            pl.BlockSpec((_BM, _C), lambda i: (i, 0)),
            pl.BlockSpec((_BM, 8), lambda i: (i, 0)),
        ],
        out_specs=pl.BlockSpec((_BM, _C), lambda i: (i, 0)),
        out_shape=jax.ShapeDtypeStruct((n, _C), jnp.float32),
    )(adj, b, prop, seed, target_base, svec)


def kernel(logits, prop_adj, struct_feat):
    n = logits.shape[0]
    seed = jax.nn.relu(logits)
    conf0 = _confidence(seed)
    weighted_seed = conf0 * seed
    global_prior = weighted_seed.sum(axis=0, keepdims=True) / jnp.clip(
        conf0.sum(), _EPS, None)
    anchor = jnp.clip(_MIN_ANCHOR + _ALPHA * conf0, 0.0, 0.995)
    uncertainty = 1.0 - conf0
    log_degree = struct_feat[:, :1]
    low_degree = jnp.clip(1.0 - log_degree, 0.0, 1.0)
    clustering = struct_feat[:, 1:2]
    low_clustering = jnp.clip(1.0 - clustering, 0.0, 1.0)
    graph_scale = jnp.clip(1.0 - clustering.mean(), 0.2, 1.0)
    struct_boost = 1.0 + _DEGREE_BIAS * low_degree + _CLUSTERING_BIAS * low_clustering

    tcoef = (1.0 - anchor) * (1.0 - _GLOBAL_BETA)
    target_base = anchor * seed + (1.0 - anchor) * _GLOBAL_BETA * global_prior
    res_coef = _RESIDUAL_SCALE * uncertainty
    rate_fixed = _GRAPH_SCALE_BIAS * graph_scale * struct_boost * uncertainty

    prop = seed
    conf = conf0
    for _ in range(_PROP_STEPS):
        source_gate = jax.nn.sigmoid(
            _SOURCE_CONF_SHARPNESS * (conf - _SOURCE_CONF_CENTER))
        recipient_gate = jax.nn.sigmoid(
            _RECIPIENT_CONF_SHARPNESS * (_RECIPIENT_CONF_CENTER - conf))
        b = jnp.concatenate(
            [source_gate * prop, source_gate,
             jnp.zeros((n, _RHS - _C - 1), dtype=jnp.float32)],
            axis=1).astype(jnp.bfloat16)

        na = jnp.maximum(
            jnp.sqrt(jnp.sum(prop * prop, axis=1, keepdims=True)), 1e-8)
        probs = prop / (prop.sum(axis=1, keepdims=True) + _EPS)
        topk, _ = jax.lax.top_k(probs, 2)
        margin = topk[:, :1] - topk[:, 1:2]
        margin_term = (_ACCEPT_MARGIN_WEIGHT * margin
                       + _ACCEPT_STRUCT_WEIGHT * clustering)
        rate_base = rate_fixed * recipient_gate

        svec = jnp.concatenate(
            [na, margin_term, rate_base, tcoef, res_coef,
             jnp.zeros((n, 3), dtype=jnp.float32)], axis=1)

        prop = _propagate_step(prop_adj, b, prop, seed, target_base, svec)
        conf = _confidence(prop)

    return prop


# speedup vs baseline: 1.3221x; 1.0140x over previous
import functools

import jax
import jax.numpy as jnp
import numpy as np
from jax.experimental import pallas as pl

_PROP_STEPS = 2
_ALPHA = 0.2
_GLOBAL_BETA = 0.05
_MIN_ANCHOR = 0.6
_RESIDUAL_SCALE = 0.15
_DEGREE_BIAS = 0.25
_CLUSTERING_BIAS = 0.2
_GRAPH_SCALE_BIAS = 1.0
_SOURCE_CONF_CENTER = 0.55
_SOURCE_CONF_SHARPNESS = 8.0
_RECIPIENT_CONF_CENTER = 0.5
_RECIPIENT_CONF_SHARPNESS = 8.0
_ACCEPT_SHARPNESS = 12.0
_ACCEPT_QUALITY_WEIGHT = 0.7
_ACCEPT_MARGIN_WEIGHT = 0.2
_ACCEPT_STRUCT_WEIGHT = 0.1
_EPS = 1e-8

_C = 64
_BM = 400
_RHS = 128


def _confidence(state):
    score_mass = state.sum(axis=1, keepdims=True)
    norm_scores = state / (score_mass + _EPS)
    max_entropy = float(np.log(state.shape[1]))
    entropy = -(norm_scores * jnp.log(norm_scores + _EPS)).sum(axis=1, keepdims=True)
    certainty = 1.0 - entropy / max_entropy
    mass_scale = jnp.clip(score_mass.mean(), _EPS, None)
    magnitude = jnp.tanh(score_mass / mass_scale)
    return jnp.clip(0.5 * certainty + 0.5 * magnitude, 0.0, 1.0)


def _step_body(adj_t_ref, adj_b_ref, b_ref, prop_ref, seed_ref,
               tb_ref, s_ref, out_ref):
    b = b_ref[...]
    acc = jnp.concatenate(
        [jnp.dot(adj_t_ref[...], b, preferred_element_type=jnp.float32),
         jnp.dot(adj_b_ref[...], b, preferred_element_type=jnp.float32)],
        axis=0)
    num = acc[:, :_C]
    den = jnp.maximum(acc[:, _C:_C + 1], _EPS)
    lc = num / den

    prop = prop_ref[...]
    na = s_ref[:, 0:1]
    margin_term = s_ref[:, 1:2]
    rate_base = s_ref[:, 2:3]
    tcoef = s_ref[:, 3:4]
    res_coef = s_ref[:, 4:5]

    dot = jnp.sum(prop * lc, axis=1, keepdims=True)
    nb = jnp.maximum(jnp.sqrt(jnp.sum(lc * lc, axis=1, keepdims=True)), 1e-8)
    local_quality = jnp.clip((dot / (na * nb) + 1.0) * 0.5, 0.0, 1.0)
    quality = _ACCEPT_QUALITY_WEIGHT * local_quality + margin_term
    accept = jax.nn.sigmoid(_ACCEPT_SHARPNESS * quality)
    step_rate = jnp.minimum(rate_base * accept, 1.0)

    target = tb_ref[...] + tcoef * lc
    p = prop + step_rate * (target - prop)
    p = p + res_coef * (seed_ref[...] - p)
    out_ref[...] = p


@functools.partial(jax.jit, static_argnames=())
def _propagate_step(adj, b, prop, seed, target_base, svec):
    n = adj.shape[0]
    hm = _BM // 2
    return pl.pallas_call(
        _step_body,
        grid=(n // _BM,),
        in_specs=[
            pl.BlockSpec((hm, n), lambda i: (2 * i, 0)),
            pl.BlockSpec((hm, n), lambda i: (2 * i + 1, 0)),
            pl.BlockSpec((n, _RHS), lambda i: (0, 0)),
            pl.BlockSpec((_BM, _C), lambda i: (i, 0)),
            pl.BlockSpec((_BM, _C), lambda i: (i, 0)),
            pl.BlockSpec((_BM, _C), lambda i: (i, 0)),
            pl.BlockSpec((_BM, 8), lambda i: (i, 0)),
        ],
        out_specs=pl.BlockSpec((_BM, _C), lambda i: (i, 0)),
        out_shape=jax.ShapeDtypeStruct((n, _C), jnp.float32),
    )(adj, adj, b, prop, seed, target_base, svec)


def kernel(logits, prop_adj, struct_feat):
    n = logits.shape[0]
    seed = jax.nn.relu(logits)
    conf0 = _confidence(seed)
    weighted_seed = conf0 * seed
    global_prior = weighted_seed.sum(axis=0, keepdims=True) / jnp.clip(
        conf0.sum(), _EPS, None)
    anchor = jnp.clip(_MIN_ANCHOR + _ALPHA * conf0, 0.0, 0.995)
    uncertainty = 1.0 - conf0
    log_degree = struct_feat[:, :1]
    low_degree = jnp.clip(1.0 - log_degree, 0.0, 1.0)
    clustering = struct_feat[:, 1:2]
    low_clustering = jnp.clip(1.0 - clustering, 0.0, 1.0)
    graph_scale = jnp.clip(1.0 - clustering.mean(), 0.2, 1.0)
    struct_boost = 1.0 + _DEGREE_BIAS * low_degree + _CLUSTERING_BIAS * low_clustering

    tcoef = (1.0 - anchor) * (1.0 - _GLOBAL_BETA)
    target_base = anchor * seed + (1.0 - anchor) * _GLOBAL_BETA * global_prior
    res_coef = _RESIDUAL_SCALE * uncertainty
    rate_fixed = _GRAPH_SCALE_BIAS * graph_scale * struct_boost * uncertainty

    prop = seed
    conf = conf0
    for _ in range(_PROP_STEPS):
        source_gate = jax.nn.sigmoid(
            _SOURCE_CONF_SHARPNESS * (conf - _SOURCE_CONF_CENTER))
        recipient_gate = jax.nn.sigmoid(
            _RECIPIENT_CONF_SHARPNESS * (_RECIPIENT_CONF_CENTER - conf))
        b = jnp.concatenate(
            [source_gate * prop, source_gate,
             jnp.zeros((n, _RHS - _C - 1), dtype=jnp.float32)],
            axis=1)

        na = jnp.maximum(
            jnp.sqrt(jnp.sum(prop * prop, axis=1, keepdims=True)), 1e-8)
        probs = prop / (prop.sum(axis=1, keepdims=True) + _EPS)
        topk, _ = jax.lax.top_k(probs, 2)
        margin = topk[:, :1] - topk[:, 1:2]
        margin_term = (_ACCEPT_MARGIN_WEIGHT * margin
                       + _ACCEPT_STRUCT_WEIGHT * clustering)
        rate_base = rate_fixed * recipient_gate

        svec = jnp.concatenate(
            [na, margin_term, rate_base, tcoef, res_coef,
             jnp.zeros((n, 3), dtype=jnp.float32)], axis=1)

        prop = _propagate_step(prop_adj, b, prop, seed, target_base, svec)
        conf = _confidence(prop)

    return prop


# speedup vs baseline: 1.4440x; 1.0921x over previous
import functools

import jax
import jax.numpy as jnp
import numpy as np
from jax.experimental import pallas as pl

_PROP_STEPS = 2
_ALPHA = 0.2
_GLOBAL_BETA = 0.05
_MIN_ANCHOR = 0.6
_RESIDUAL_SCALE = 0.15
_DEGREE_BIAS = 0.25
_CLUSTERING_BIAS = 0.2
_GRAPH_SCALE_BIAS = 1.0
_SOURCE_CONF_CENTER = 0.55
_SOURCE_CONF_SHARPNESS = 8.0
_RECIPIENT_CONF_CENTER = 0.5
_RECIPIENT_CONF_SHARPNESS = 8.0
_ACCEPT_SHARPNESS = 12.0
_ACCEPT_QUALITY_WEIGHT = 0.7
_ACCEPT_MARGIN_WEIGHT = 0.2
_ACCEPT_STRUCT_WEIGHT = 0.1
_EPS = 1e-8

_C = 64
_BM = 400
_RHS = 128


def _confidence(state):
    score_mass = state.sum(axis=1, keepdims=True)
    norm_scores = state / (score_mass + _EPS)
    max_entropy = float(np.log(state.shape[1]))
    entropy = -(norm_scores * jnp.log(norm_scores + _EPS)).sum(axis=1, keepdims=True)
    certainty = 1.0 - entropy / max_entropy
    mass_scale = jnp.clip(score_mass.mean(), _EPS, None)
    magnitude = jnp.tanh(score_mass / mass_scale)
    return jnp.clip(0.5 * certainty + 0.5 * magnitude, 0.0, 1.0)


def _epilogue(acc, prop_ref, seed_ref, tb_ref, s_ref, out_ref):
    num = acc[:, :_C]
    den = jnp.maximum(acc[:, _C:_C + 1], _EPS)
    lc = num / den

    prop = prop_ref[...]
    na = s_ref[:, 0:1]
    margin_term = s_ref[:, 1:2]
    rate_base = s_ref[:, 2:3]
    tcoef = s_ref[:, 3:4]
    res_coef = s_ref[:, 4:5]

    dot = jnp.sum(prop * lc, axis=1, keepdims=True)
    nb = jnp.maximum(jnp.sqrt(jnp.sum(lc * lc, axis=1, keepdims=True)), 1e-8)
    local_quality = jnp.clip((dot / (na * nb) + 1.0) * 0.5, 0.0, 1.0)
    quality = _ACCEPT_QUALITY_WEIGHT * local_quality + margin_term
    accept = jax.nn.sigmoid(_ACCEPT_SHARPNESS * quality)
    step_rate = jnp.minimum(rate_base * accept, 1.0)

    target = tb_ref[...] + tcoef * lc
    p = prop + step_rate * (target - prop)
    p = p + res_coef * (seed_ref[...] - p)
    out_ref[...] = p


def _step1_body(adj_t_ref, adj_b_ref, b_ref, prop_ref, seed_ref,
                tb_ref, s_ref, out_ref, outq_ref):
    b = b_ref[...]
    at = adj_t_ref[...]
    ab = adj_b_ref[...]
    acc = jnp.concatenate(
        [jnp.dot(at, b, preferred_element_type=jnp.float32),
         jnp.dot(ab, b, preferred_element_type=jnp.float32)],
        axis=0)
    hm = at.shape[0]
    outq_ref[:hm, :] = at.astype(jnp.float8_e4m3fn)
    outq_ref[hm:, :] = ab.astype(jnp.float8_e4m3fn)
    _epilogue(acc, prop_ref, seed_ref, tb_ref, s_ref, out_ref)


def _step2_body(adj_t_ref, adj_b_ref, b_ref, prop_ref, seed_ref,
                tb_ref, s_ref, out_ref):
    b = b_ref[...]
    acc = jnp.concatenate(
        [jnp.dot(adj_t_ref[...], b, preferred_element_type=jnp.float32),
         jnp.dot(adj_b_ref[...], b, preferred_element_type=jnp.float32)],
        axis=0)
    _epilogue(acc, prop_ref, seed_ref, tb_ref, s_ref, out_ref)


def _row_specs(n):
    hm = _BM // 2
    return [
        pl.BlockSpec((hm, n), lambda i: (2 * i, 0)),
        pl.BlockSpec((hm, n), lambda i: (2 * i + 1, 0)),
        pl.BlockSpec((n, _RHS), lambda i: (0, 0)),
        pl.BlockSpec((_BM, _C), lambda i: (i, 0)),
        pl.BlockSpec((_BM, _C), lambda i: (i, 0)),
        pl.BlockSpec((_BM, _C), lambda i: (i, 0)),
        pl.BlockSpec((_BM, 8), lambda i: (i, 0)),
    ]


def _propagate_step1(adj, b, prop, seed, target_base, svec):
    n = adj.shape[0]
    return pl.pallas_call(
        _step1_body,
        grid=(n // _BM,),
        in_specs=_row_specs(n),
        out_specs=[
            pl.BlockSpec((_BM, _C), lambda i: (i, 0)),
            pl.BlockSpec((_BM, n), lambda i: (i, 0)),
        ],
        out_shape=[
            jax.ShapeDtypeStruct((n, _C), jnp.float32),
            jax.ShapeDtypeStruct((n, n), jnp.float8_e4m3fn),
        ],
    )(adj, adj, b, prop, seed, target_base, svec)


def _propagate_step2(adj_q, b, prop, seed, target_base, svec):
    n = adj_q.shape[0]
    return pl.pallas_call(
        _step2_body,
        grid=(n // _BM,),
        in_specs=_row_specs(n),
        out_specs=pl.BlockSpec((_BM, _C), lambda i: (i, 0)),
        out_shape=jax.ShapeDtypeStruct((n, _C), jnp.float32),
    )(adj_q, adj_q, b, prop, seed, target_base, svec)


def kernel(logits, prop_adj, struct_feat):
    n = logits.shape[0]
    seed = jax.nn.relu(logits)
    conf0 = _confidence(seed)
    weighted_seed = conf0 * seed
    global_prior = weighted_seed.sum(axis=0, keepdims=True) / jnp.clip(
        conf0.sum(), _EPS, None)
    anchor = jnp.clip(_MIN_ANCHOR + _ALPHA * conf0, 0.0, 0.995)
    uncertainty = 1.0 - conf0
    log_degree = struct_feat[:, :1]
    low_degree = jnp.clip(1.0 - log_degree, 0.0, 1.0)
    clustering = struct_feat[:, 1:2]
    low_clustering = jnp.clip(1.0 - clustering, 0.0, 1.0)
    graph_scale = jnp.clip(1.0 - clustering.mean(), 0.2, 1.0)
    struct_boost = 1.0 + _DEGREE_BIAS * low_degree + _CLUSTERING_BIAS * low_clustering

    tcoef = (1.0 - anchor) * (1.0 - _GLOBAL_BETA)
    target_base = anchor * seed + (1.0 - anchor) * _GLOBAL_BETA * global_prior
    res_coef = _RESIDUAL_SCALE * uncertainty
    rate_fixed = _GRAPH_SCALE_BIAS * graph_scale * struct_boost * uncertainty

    def _side_inputs(prop, conf):
        source_gate = jax.nn.sigmoid(
            _SOURCE_CONF_SHARPNESS * (conf - _SOURCE_CONF_CENTER))
        recipient_gate = jax.nn.sigmoid(
            _RECIPIENT_CONF_SHARPNESS * (_RECIPIENT_CONF_CENTER - conf))
        b = jnp.concatenate(
            [source_gate * prop, source_gate,
             jnp.zeros((n, _RHS - _C - 1), dtype=jnp.float32)],
            axis=1)
        na = jnp.maximum(
            jnp.sqrt(jnp.sum(prop * prop, axis=1, keepdims=True)), 1e-8)
        probs = prop / (prop.sum(axis=1, keepdims=True) + _EPS)
        topk, _ = jax.lax.top_k(probs, 2)
        margin = topk[:, :1] - topk[:, 1:2]
        margin_term = (_ACCEPT_MARGIN_WEIGHT * margin
                       + _ACCEPT_STRUCT_WEIGHT * clustering)
        rate_base = rate_fixed * recipient_gate
        svec = jnp.concatenate(
            [na, margin_term, rate_base, tcoef, res_coef,
             jnp.zeros((n, 3), dtype=jnp.float32)], axis=1)
        return b, svec

    b, svec = _side_inputs(seed, conf0)
    prop, adj_q = _propagate_step1(prop_adj, b, seed, seed, target_base, svec)
    conf = _confidence(prop)
    b, svec = _side_inputs(prop, conf)
    prop = _propagate_step2(adj_q, b.astype(jnp.float8_e4m3fn), prop, seed,
                            target_base, svec)
    return prop
